# Initial kernel scaffold; baseline (speedup 1.0000x reference)
import jax
import jax.numpy as jnp
from jax.experimental import pallas as pl


def kernel(features, edges, W_self, W_neigh, bias):
    raise NotImplementedError("write your pallas kernel here")


# speedup vs baseline: 3.8625x; 3.8625x over previous
import functools

import jax
import jax.numpy as jnp
from jax import lax
from jax.experimental import pallas as pl
from jax.experimental.pallas import tpu as pltpu
from jax.experimental.pallas import tpu_sc as plsc

_N = 10000
_E = 320000
_D = 128
_NC = 2
_NS = 16
_NW = _NC * _NS
_EB = 128
_NBLK = -(-_E // (_NW * _EB))
_EP = _NW * _EB * _NBLK
_STRIPE = 632
_AR = _STRIPE * _NS


def _sc_segment_sum(src_r, dst_r, x, zrows):
    mesh = plsc.VectorSubcoreMesh(core_axis_name="c", subcore_axis_name="s")

    @functools.partial(
        pl.kernel,
        mesh=mesh,
        out_type=jax.ShapeDtypeStruct((_NC, _AR, _D), jnp.float32),
        scratch_types=[
            pltpu.VMEM((1, _EB), jnp.int32),
            pltpu.VMEM((1, _EB), jnp.int32),
            pltpu.VMEM((_EB, _D), jnp.float32),
            pltpu.VMEM_SHARED((_AR, _D), jnp.float32),
            pltpu.SemaphoreType.DMA,
        ],
    )
    def k(src_hbm, dst_hbm, x_hbm, z_hbm, out_hbm, sidx, didx, rows, acc, sem):
        cid = lax.axis_index("c")
        sid = lax.axis_index("s")
        wid = cid * _NS + sid
        pltpu.sync_copy(z_hbm, acc.at[pl.ds(sid * _STRIPE, _STRIPE)])
        plsc.subcore_barrier()

        def body(blk, carry):
            pltpu.sync_copy(src_hbm.at[wid, blk], sidx.at[0])
            pltpu.sync_copy(dst_hbm.at[wid, blk], didx.at[0])
            pltpu.async_copy(x_hbm.at[sidx.at[0]], rows, sem).wait()
            pltpu.sync_copy(rows, acc.at[didx.at[0]], add=True)
            return carry

        lax.fori_loop(0, _NBLK, body, 0)
        plsc.subcore_barrier()
        pltpu.sync_copy(acc.at[pl.ds(sid * _STRIPE, _STRIPE)],
                        out_hbm.at[cid, pl.ds(sid * _STRIPE, _STRIPE)])

    return k(src_r, dst_r, x, zrows)


_BLK = 1000


def _dense_body(x_ref, p_ref, ws_ref, wn_ref, b_ref, o_ref):
    agg = p_ref[0] + p_ref[1]
    acc = jnp.dot(x_ref[...], ws_ref[...], preferred_element_type=jnp.float32)
    acc = acc + jnp.dot(agg, wn_ref[...], preferred_element_type=jnp.float32)
    o_ref[...] = jnp.maximum(acc + b_ref[...], 0.0)


def _dense_res_body(x_ref, p_ref, ws_ref, wn_ref, b_ref, r_ref, o_ref):
    agg = p_ref[0] + p_ref[1]
    acc = jnp.dot(x_ref[...], ws_ref[...], preferred_element_type=jnp.float32)
    acc = acc + jnp.dot(agg, wn_ref[...], preferred_element_type=jnp.float32)
    o_ref[...] = jnp.maximum(acc + b_ref[...] + r_ref[...], 0.0)


def _dense(x, parts, Ws, Wn, b, res=None):
    nblk = _N // _BLK
    specs = [
        pl.BlockSpec((_BLK, _D), lambda i: (i, 0)),
        pl.BlockSpec((_NC, _BLK, _D), lambda i: (0, i, 0)),
        pl.BlockSpec((_D, _D), lambda i: (0, 0)),
        pl.BlockSpec((_D, _D), lambda i: (0, 0)),
        pl.BlockSpec((1, _D), lambda i: (0, 0)),
    ]
    args = [x, parts, Ws, Wn, b.reshape(1, _D)]
    body = _dense_body
    if res is not None:
        specs.append(pl.BlockSpec((_BLK, _D), lambda i: (i, 0)))
        args.append(res)
        body = _dense_res_body
    return pl.pallas_call(
        body,
        grid=(nblk,),
        in_specs=specs,
        out_specs=pl.BlockSpec((_BLK, _D), lambda i: (i, 0)),
        out_shape=jax.ShapeDtypeStruct((_N, _D), jnp.float32),
    )(*args)


def kernel(features, edges, W_self, W_neigh, bias):
    src = edges[0].astype(jnp.int32)
    dst = edges[1].astype(jnp.int32)
    pad = _EP - _E
    srcp = jnp.concatenate([src, jnp.zeros((pad,), jnp.int32)]).reshape(
        _NW, _NBLK, _EB)
    dstp = jnp.concatenate([dst, jnp.full((pad,), _N, jnp.int32)]).reshape(
        _NW, _NBLK, _EB)
    zrows = jnp.zeros((_STRIPE, _D), jnp.float32)
    h = features
    for i in range(3):
        parts = _sc_segment_sum(srcp, dstp, h, zrows)
        h = _dense(h, parts, W_self[i], W_neigh[i], bias[i],
                   features if i == 2 else None)
    return h
